# Initial kernel scaffold; baseline (speedup 1.0000x reference)
import jax
import jax.numpy as jnp
from jax.experimental import pallas as pl


def kernel(embedding, eclass_ids, enode_ids, W1, b1, gamma, beta, W2, b2):
    raise NotImplementedError("write your pallas kernel here")


# speedup vs baseline: 16.5817x; 16.5817x over previous
import functools

import jax
import jax.numpy as jnp
from jax import lax
from jax.experimental import pallas as pl
from jax.experimental.pallas import tpu as pltpu
from jax.experimental.pallas import tpu_sc as plsc

N_ENODES = 100000
N_ECLASSES = 50000
N_EDGES = 1600000
HIDDEN = 32

NC, NS = 2, 16
CHUNK = 125
ROWS_TOTAL = N_EDGES // CHUNK
ROWS_PER_SC = ROWS_TOTAL // NC
ROWS_PER_TILE = ROWS_PER_SC // NS
ROWS_PER_SLAB = 16
SLABS = ROWS_PER_TILE // ROWS_PER_SLAB
GROUPS = 4
ROWS_PER_GROUP = ROWS_PER_SLAB // GROUPS
ACC_PAD = 50048
ACC_STRIPE = ACC_PAD // NS
CNT_PAD = ACC_PAD


def _dense_body(emb_ref, w1_ref, b1_ref, g_ref, be_ref, w2_ref, b2_ref,
                h_ref, logit_ref):
    x = emb_ref[...]
    h = jnp.dot(x, w1_ref[...], preferred_element_type=jnp.float32) + b1_ref[...]
    mu = jnp.mean(h, axis=-1, keepdims=True)
    var = jnp.mean((h - mu) ** 2, axis=-1, keepdims=True)
    h = g_ref[...] * (h - mu) * lax.rsqrt(var + 1e-5) + be_ref[...]
    h = jnp.maximum(h, 0.0)
    h_ref[...] = h
    logit_ref[...] = jnp.sum(h * w2_ref[...], axis=-1, keepdims=True) + b2_ref[...]


def _dense(emb, W1, b1, gamma, beta, W2, b2):
    R = 10000
    grid = (N_ENODES // R,)
    h, logit = pl.pallas_call(
        _dense_body,
        grid=grid,
        in_specs=[
            pl.BlockSpec((R, HIDDEN), lambda i: (i, 0)),
            pl.BlockSpec((HIDDEN, HIDDEN), lambda i: (0, 0)),
            pl.BlockSpec((1, HIDDEN), lambda i: (0, 0)),
            pl.BlockSpec((1, HIDDEN), lambda i: (0, 0)),
            pl.BlockSpec((1, HIDDEN), lambda i: (0, 0)),
            pl.BlockSpec((1, HIDDEN), lambda i: (0, 0)),
            pl.BlockSpec((1, 1), lambda i: (0, 0)),
        ],
        out_specs=[
            pl.BlockSpec((R, HIDDEN), lambda i: (i, 0)),
            pl.BlockSpec((R, 1), lambda i: (i, 0)),
        ],
        out_shape=[
            jax.ShapeDtypeStruct((N_ENODES, HIDDEN), jnp.float32),
            jax.ShapeDtypeStruct((N_ENODES, 1), jnp.float32),
        ],
    )(emb, W1, b1.reshape(1, -1), gamma.reshape(1, -1), beta.reshape(1, -1),
      W2.reshape(1, -1), b2.reshape(1, 1))
    return h, logit


def _sc_body(h_ref, enode_ref, ecl_ref, z2d_ref, z1d_ref, acc_out, cnt_out,
             idx_slab, ecl_slab, rows, ones_v, sem,
             acc_sp, cnt_sp):
    c = lax.axis_index("c")
    s = lax.axis_index("s")

    o16 = jnp.ones((16,), jnp.float32)
    for k in range(8):
        ones_v[pl.ds(k * 16, 16)] = o16

    if True:
        pltpu.sync_copy(z2d_ref, acc_sp.at[pl.ds(s * ACC_STRIPE, ACC_STRIPE)])
        pltpu.sync_copy(z1d_ref, cnt_sp.at[pl.ds(s * ACC_STRIPE, ACC_STRIPE)])
        plsc.subcore_barrier()

        tile_row0 = c * ROWS_PER_SC + s * ROWS_PER_TILE

        def _slab(si, carry):
            row0 = tile_row0 + si * ROWS_PER_SLAB
            pltpu.sync_copy(enode_ref.at[pl.ds(row0, ROWS_PER_SLAB)], idx_slab)
            pltpu.sync_copy(ecl_ref.at[pl.ds(row0, ROWS_PER_SLAB)], ecl_slab)

            def _group(g, gc):
                r0 = g * ROWS_PER_GROUP
                descs = []
                for j in range(ROWS_PER_GROUP):
                    descs.append(pltpu.async_copy(
                        h_ref.at[idx_slab.at[r0 + j]], rows.at[j], sem))
                for d in descs:
                    d.wait()
                for j in range(ROWS_PER_GROUP):
                    pltpu.sync_copy(rows.at[j],
                                    acc_sp.at[ecl_slab.at[r0 + j]], add=True)
                    pltpu.sync_copy(ones_v.at[pl.ds(0, CHUNK)],
                                    cnt_sp.at[ecl_slab.at[r0 + j]], add=True)
                return gc
            lax.fori_loop(0, GROUPS, _group, 0)
            return carry
        lax.fori_loop(0, SLABS, _slab, 0)

        plsc.subcore_barrier()

        r = s * ACC_STRIPE
        pltpu.sync_copy(acc_sp.at[pl.ds(r, ACC_STRIPE)],
                        acc_out.at[c, pl.ds(r, ACC_STRIPE)])
        pltpu.sync_copy(cnt_sp.at[pl.ds(r, ACC_STRIPE)],
                        cnt_out.at[c, pl.ds(r, ACC_STRIPE)])


def _segment_mean_partials(h, enode2d, ecl2d, z2d, z1d):
    mesh = plsc.VectorSubcoreMesh(core_axis_name="c", subcore_axis_name="s",
                                  num_cores=NC, num_subcores=NS)
    fn = pl.kernel(
        _sc_body,
        out_type=[
            jax.ShapeDtypeStruct((NC, ACC_PAD, HIDDEN), jnp.float32),
            jax.ShapeDtypeStruct((NC, CNT_PAD), jnp.float32),
        ],
        mesh=mesh,
        scratch_types=[
            pltpu.VMEM((ROWS_PER_SLAB, CHUNK), jnp.int32),
            pltpu.VMEM((ROWS_PER_SLAB, CHUNK), jnp.int32),
            pltpu.VMEM((ROWS_PER_GROUP, CHUNK, HIDDEN), jnp.float32),
            pltpu.VMEM((128,), jnp.float32),
            pltpu.SemaphoreType.DMA,
            pltpu.VMEM_SHARED((ACC_PAD, HIDDEN), jnp.float32),
            pltpu.VMEM_SHARED((CNT_PAD,), jnp.float32),
        ],
        compiler_params=pltpu.CompilerParams(use_tc_tiling_on_sc=False),
    )
    return fn(h, enode2d, ecl2d, z2d, z1d)


def _combine_body(acc_ref, cnt_ref, ctx_ref):
    a = acc_ref[0] + acc_ref[1]
    n = jnp.sum(cnt_ref[...], axis=0)
    ctx_ref[...] = a * (1.0 / jnp.maximum(n, 1.0))


def _combine(acc, cnt):
    R = 5000
    grid = (N_ECLASSES // R,)
    return pl.pallas_call(
        _combine_body,
        grid=grid,
        in_specs=[
            pl.BlockSpec((NC, R, HIDDEN), lambda i: (0, i, 0)),
            pl.BlockSpec((NC, R, 1), lambda i: (0, i, 0)),
        ],
        out_specs=pl.BlockSpec((R, HIDDEN), lambda i: (i, 0)),
        out_shape=jax.ShapeDtypeStruct((N_ECLASSES, HIDDEN), jnp.float32),
    )(acc, cnt)


def kernel(embedding, eclass_ids, enode_ids, W1, b1, gamma, beta, W2, b2):
    emb = embedding.reshape(N_ENODES, HIDDEN)
    h, logit = _dense(emb, W1, b1, gamma, beta, W2, b2)
    enode2d = enode_ids.reshape(ROWS_TOTAL, CHUNK)
    ecl2d = eclass_ids.reshape(ROWS_TOTAL, CHUNK)
    z2d = jnp.zeros((ACC_STRIPE, HIDDEN), jnp.float32)
    z1d = jnp.zeros((ACC_STRIPE,), jnp.float32)
    acc, cnt = _segment_mean_partials(h, enode2d, ecl2d, z2d, z1d)
    ctx = _combine(acc[:, :N_ECLASSES],
                   cnt[:, :N_ECLASSES].reshape(NC, N_ECLASSES, 1))
    return logit.reshape(1, N_ENODES), ctx.reshape(1, N_ECLASSES, HIDDEN)


# speedup vs baseline: 18.5049x; 1.1160x over previous
import functools

import jax
import jax.numpy as jnp
from jax import lax
from jax.experimental import pallas as pl
from jax.experimental.pallas import tpu as pltpu
from jax.experimental.pallas import tpu_sc as plsc

N_ENODES = 100000
N_ECLASSES = 50000
N_EDGES = 1600000
HIDDEN = 32

NC, NS = 2, 16
CHUNK = 250
ROWS_TOTAL = N_EDGES // CHUNK
ROWS_PER_SC = ROWS_TOTAL // NC
ROWS_PER_TILE = ROWS_PER_SC // NS
ROWS_PER_SLAB = 8
SLABS = ROWS_PER_TILE // ROWS_PER_SLAB
ACC_PAD = 50048
ACC_STRIPE = ACC_PAD // NS
CNT_PAD = ACC_PAD


def _dense_body(emb_ref, w1_ref, b1_ref, g_ref, be_ref, w2_ref, b2_ref,
                h_ref, logit_ref):
    x = emb_ref[...]
    h = jnp.dot(x, w1_ref[...], preferred_element_type=jnp.float32) + b1_ref[...]
    mu = jnp.mean(h, axis=-1, keepdims=True)
    var = jnp.mean((h - mu) ** 2, axis=-1, keepdims=True)
    h = g_ref[...] * (h - mu) * lax.rsqrt(var + 1e-5) + be_ref[...]
    h = jnp.maximum(h, 0.0)
    h_ref[...] = h
    logit_ref[...] = jnp.sum(h * w2_ref[...], axis=-1, keepdims=True) + b2_ref[...]


def _dense(emb, W1, b1, gamma, beta, W2, b2):
    R = 10000
    grid = (N_ENODES // R,)
    h, logit = pl.pallas_call(
        _dense_body,
        grid=grid,
        in_specs=[
            pl.BlockSpec((R, HIDDEN), lambda i: (i, 0)),
            pl.BlockSpec((HIDDEN, HIDDEN), lambda i: (0, 0)),
            pl.BlockSpec((1, HIDDEN), lambda i: (0, 0)),
            pl.BlockSpec((1, HIDDEN), lambda i: (0, 0)),
            pl.BlockSpec((1, HIDDEN), lambda i: (0, 0)),
            pl.BlockSpec((1, HIDDEN), lambda i: (0, 0)),
            pl.BlockSpec((1, 1), lambda i: (0, 0)),
        ],
        out_specs=[
            pl.BlockSpec((R, HIDDEN), lambda i: (i, 0)),
            pl.BlockSpec((R, 1), lambda i: (i, 0)),
        ],
        out_shape=[
            jax.ShapeDtypeStruct((N_ENODES, HIDDEN), jnp.float32),
            jax.ShapeDtypeStruct((N_ENODES, 1), jnp.float32),
        ],
    )(emb, W1, b1.reshape(1, -1), gamma.reshape(1, -1), beta.reshape(1, -1),
      W2.reshape(1, -1), b2.reshape(1, 1))
    return h, logit


def _sc_body(h_ref, enode_ref, ecl_ref, z2d_ref, z1d_ref, ones_ref,
             acc_out, cnt_out,
             idx_slab, ecl_slab, rows0, rows1, ones_v, sem0, sem1,
             acc_sp, cnt_sp):
    c = lax.axis_index("c")
    s = lax.axis_index("s")

    pltpu.sync_copy(ones_ref, ones_v)

    if True:
        pltpu.sync_copy(z2d_ref, acc_sp.at[pl.ds(s * ACC_STRIPE, ACC_STRIPE)])
        pltpu.sync_copy(z1d_ref, cnt_sp.at[pl.ds(s * ACC_STRIPE, ACC_STRIPE)])
        plsc.subcore_barrier()

        tile_row0 = c * ROWS_PER_SC + s * ROWS_PER_TILE

        bufs = (rows0, rows1)
        sems = (sem0, sem1)

        def _slab(si, carry):
            row0 = tile_row0 + si * ROWS_PER_SLAB
            pltpu.sync_copy(enode_ref.at[pl.ds(row0, ROWS_PER_SLAB)], idx_slab)
            pltpu.sync_copy(ecl_ref.at[pl.ds(row0, ROWS_PER_SLAB)], ecl_slab)

            descs = [None, None]
            descs[0] = pltpu.async_copy(h_ref.at[idx_slab.at[0]], bufs[0], sems[0])
            for j in range(ROWS_PER_SLAB):
                p = j % 2
                if j + 1 < ROWS_PER_SLAB:
                    descs[1 - p] = pltpu.async_copy(
                        h_ref.at[idx_slab.at[j + 1]], bufs[1 - p], sems[1 - p])
                descs[p].wait()
                pltpu.sync_copy(bufs[p], acc_sp.at[ecl_slab.at[j]], add=True)
                pltpu.sync_copy(ones_v, cnt_sp.at[ecl_slab.at[j]], add=True)
            return carry
        lax.fori_loop(0, SLABS, _slab, 0)

        plsc.subcore_barrier()

        r = s * ACC_STRIPE
        pltpu.sync_copy(acc_sp.at[pl.ds(r, ACC_STRIPE)],
                        acc_out.at[c, pl.ds(r, ACC_STRIPE)])
        pltpu.sync_copy(cnt_sp.at[pl.ds(r, ACC_STRIPE)],
                        cnt_out.at[c, pl.ds(r, ACC_STRIPE)])


def _segment_mean_partials(h, enode2d, ecl2d, z2d, z1d, ones2d):
    mesh = plsc.VectorSubcoreMesh(core_axis_name="c", subcore_axis_name="s",
                                  num_cores=NC, num_subcores=NS)
    fn = pl.kernel(
        _sc_body,
        out_type=[
            jax.ShapeDtypeStruct((NC, ACC_PAD, HIDDEN), jnp.float32),
            jax.ShapeDtypeStruct((NC, CNT_PAD), jnp.float32),
        ],
        mesh=mesh,
        scratch_types=[
            pltpu.VMEM((ROWS_PER_SLAB, CHUNK), jnp.int32),
            pltpu.VMEM((ROWS_PER_SLAB, CHUNK), jnp.int32),
            pltpu.VMEM((CHUNK, HIDDEN), jnp.float32),
            pltpu.VMEM((CHUNK, HIDDEN), jnp.float32),
            pltpu.VMEM((CHUNK,), jnp.float32),
            pltpu.SemaphoreType.DMA,
            pltpu.SemaphoreType.DMA,
            pltpu.VMEM_SHARED((ACC_PAD, HIDDEN), jnp.float32),
            pltpu.VMEM_SHARED((CNT_PAD,), jnp.float32),
        ],
        compiler_params=pltpu.CompilerParams(use_tc_tiling_on_sc=False),
    )
    return fn(h, enode2d, ecl2d, z2d, z1d, ones2d)


def _combine_body(acc_ref, cnt_ref, ctx_ref):
    a = acc_ref[0] + acc_ref[1]
    n = jnp.sum(cnt_ref[...], axis=0)
    ctx_ref[...] = a * (1.0 / jnp.maximum(n, 1.0))


def _combine(acc, cnt):
    R = 5000
    grid = (N_ECLASSES // R,)
    return pl.pallas_call(
        _combine_body,
        grid=grid,
        in_specs=[
            pl.BlockSpec((NC, R, HIDDEN), lambda i: (0, i, 0)),
            pl.BlockSpec((NC, R, 1), lambda i: (0, i, 0)),
        ],
        out_specs=pl.BlockSpec((R, HIDDEN), lambda i: (i, 0)),
        out_shape=jax.ShapeDtypeStruct((N_ECLASSES, HIDDEN), jnp.float32),
    )(acc, cnt)


def kernel(embedding, eclass_ids, enode_ids, W1, b1, gamma, beta, W2, b2):
    emb = embedding.reshape(N_ENODES, HIDDEN)
    h, logit = _dense(emb, W1, b1, gamma, beta, W2, b2)
    enode2d = enode_ids.reshape(ROWS_TOTAL, CHUNK)
    ecl2d = eclass_ids.reshape(ROWS_TOTAL, CHUNK)
    z2d = jnp.zeros((ACC_STRIPE, HIDDEN), jnp.float32)
    z1d = jnp.zeros((ACC_STRIPE,), jnp.float32)
    ones2d = jnp.ones((CHUNK,), jnp.float32)
    acc, cnt = _segment_mean_partials(h, enode2d, ecl2d, z2d, z1d, ones2d)
    ctx = _combine(acc[:, :N_ECLASSES],
                   cnt[:, :N_ECLASSES].reshape(NC, N_ECLASSES, 1))
    return logit.reshape(1, N_ENODES), ctx.reshape(1, N_ECLASSES, HIDDEN)


# speedup vs baseline: 23.1803x; 1.2527x over previous
import functools

import jax
import jax.numpy as jnp
from jax import lax
from jax.experimental import pallas as pl
from jax.experimental.pallas import tpu as pltpu
from jax.experimental.pallas import tpu_sc as plsc

N_ENODES = 100000
N_ECLASSES = 50000
N_EDGES = 1600000
HIDDEN = 32

NC, NS = 2, 16
CHUNK = 250
ROWS_TOTAL = N_EDGES // CHUNK
ROWS_PER_SC = ROWS_TOTAL // NC
ROWS_PER_TILE = ROWS_PER_SC // NS
ROWS_PER_SLAB = 8
SLABS = ROWS_PER_TILE // ROWS_PER_SLAB
ACC_PAD = 50048
ACC_STRIPE = ACC_PAD // NS
CNT_PAD = ACC_PAD


PACK = 128 // HIDDEN
NP4 = N_ENODES // PACK


def _dense_body(emb_ref, w1b_ref, b1b_ref, gb_ref, beb_ref, w2b_ref, b2_ref,
                m_ref, mt_ref, h_ref, logit_ref):
    x = emb_ref[...]
    h = jnp.dot(x, w1b_ref[...], preferred_element_type=jnp.float32) + b1b_ref[...]
    m, mt = m_ref[...], mt_ref[...]
    mu4 = jnp.dot(h, m, preferred_element_type=jnp.float32) * (1.0 / HIDDEN)
    d = h - jnp.dot(mu4, mt, preferred_element_type=jnp.float32)
    var4 = jnp.dot(d * d, m, preferred_element_type=jnp.float32) * (1.0 / HIDDEN)
    var = jnp.dot(var4, mt, preferred_element_type=jnp.float32)
    h = gb_ref[...] * d * lax.rsqrt(var + 1e-5) + beb_ref[...]
    h = jnp.maximum(h, 0.0)
    h_ref[...] = h
    logit_ref[...] = (jnp.dot(h, w2b_ref[...], preferred_element_type=jnp.float32)
                      + b2_ref[...])


def _dense(emb4, W1, b1, gamma, beta, W2, b2):
    R = 5000
    grid = (NP4 // R,)
    eye4 = jnp.eye(PACK, dtype=jnp.float32)
    W1b = jnp.kron(eye4, W1)
    W2b = jnp.kron(eye4, W2)
    M = jnp.kron(eye4, jnp.ones((HIDDEN, 1), jnp.float32))
    b1b = jnp.tile(b1, PACK).reshape(1, -1)
    gb = jnp.tile(gamma, PACK).reshape(1, -1)
    beb = jnp.tile(beta, PACK).reshape(1, -1)
    h, logit = pl.pallas_call(
        _dense_body,
        grid=grid,
        in_specs=[
            pl.BlockSpec((R, 128), lambda i: (i, 0)),
            pl.BlockSpec((128, 128), lambda i: (0, 0)),
            pl.BlockSpec((1, 128), lambda i: (0, 0)),
            pl.BlockSpec((1, 128), lambda i: (0, 0)),
            pl.BlockSpec((1, 128), lambda i: (0, 0)),
            pl.BlockSpec((128, PACK), lambda i: (0, 0)),
            pl.BlockSpec((1, 1), lambda i: (0, 0)),
            pl.BlockSpec((128, PACK), lambda i: (0, 0)),
            pl.BlockSpec((PACK, 128), lambda i: (0, 0)),
        ],
        out_specs=[
            pl.BlockSpec((R, 128), lambda i: (i, 0)),
            pl.BlockSpec((R, PACK), lambda i: (i, 0)),
        ],
        out_shape=[
            jax.ShapeDtypeStruct((NP4, 128), jnp.float32),
            jax.ShapeDtypeStruct((NP4, PACK), jnp.float32),
        ],
    )(emb4, W1b, b1b, gb, beb, W2b, b2.reshape(1, 1), M, M.T)
    return h, logit


def _sc_body(h_ref, enode_ref, ecl_ref, z2d_ref, z1d_ref, ones_ref,
             acc_out, cnt_out,
             idx_slab, ecl_slab, rows0, rows1, ones_v, sem0, sem1,
             acc_sp, cnt_sp):
    c = lax.axis_index("c")
    s = lax.axis_index("s")

    pltpu.sync_copy(ones_ref, ones_v)

    if True:
        pltpu.sync_copy(z2d_ref, acc_sp.at[pl.ds(s * ACC_STRIPE, ACC_STRIPE)])
        pltpu.sync_copy(z1d_ref, cnt_sp.at[pl.ds(s * ACC_STRIPE, ACC_STRIPE)])
        plsc.subcore_barrier()

        tile_row0 = c * ROWS_PER_SC + s * ROWS_PER_TILE

        bufs = (rows0, rows1)
        sems = (sem0, sem1)

        def _slab(si, carry):
            row0 = tile_row0 + si * ROWS_PER_SLAB
            pltpu.sync_copy(enode_ref.at[pl.ds(row0, ROWS_PER_SLAB)], idx_slab)
            pltpu.sync_copy(ecl_ref.at[pl.ds(row0, ROWS_PER_SLAB)], ecl_slab)

            descs = [None, None]
            descs[0] = pltpu.async_copy(h_ref.at[idx_slab.at[0]], bufs[0], sems[0])
            for j in range(ROWS_PER_SLAB):
                p = j % 2
                if j + 1 < ROWS_PER_SLAB:
                    descs[1 - p] = pltpu.async_copy(
                        h_ref.at[idx_slab.at[j + 1]], bufs[1 - p], sems[1 - p])
                descs[p].wait()
                pltpu.sync_copy(bufs[p], acc_sp.at[ecl_slab.at[j]], add=True)
                pltpu.sync_copy(ones_v, cnt_sp.at[ecl_slab.at[j]], add=True)
            return carry
        lax.fori_loop(0, SLABS, _slab, 0)

        plsc.subcore_barrier()

        r = s * ACC_STRIPE
        pltpu.sync_copy(acc_sp.at[pl.ds(r, ACC_STRIPE)],
                        acc_out.at[c, pl.ds(r, ACC_STRIPE)])
        pltpu.sync_copy(cnt_sp.at[pl.ds(r, ACC_STRIPE)],
                        cnt_out.at[c, pl.ds(r, ACC_STRIPE)])


def _segment_mean_partials(h, enode2d, ecl2d, z2d, z1d, ones2d):
    mesh = plsc.VectorSubcoreMesh(core_axis_name="c", subcore_axis_name="s",
                                  num_cores=NC, num_subcores=NS)
    fn = pl.kernel(
        _sc_body,
        out_type=[
            jax.ShapeDtypeStruct((NC, ACC_PAD, HIDDEN), jnp.float32),
            jax.ShapeDtypeStruct((NC, CNT_PAD), jnp.float32),
        ],
        mesh=mesh,
        scratch_types=[
            pltpu.VMEM((ROWS_PER_SLAB, CHUNK), jnp.int32),
            pltpu.VMEM((ROWS_PER_SLAB, CHUNK), jnp.int32),
            pltpu.VMEM((CHUNK, HIDDEN), jnp.float32),
            pltpu.VMEM((CHUNK, HIDDEN), jnp.float32),
            pltpu.VMEM((CHUNK,), jnp.float32),
            pltpu.SemaphoreType.DMA,
            pltpu.SemaphoreType.DMA,
            pltpu.VMEM_SHARED((ACC_PAD, HIDDEN), jnp.float32),
            pltpu.VMEM_SHARED((CNT_PAD,), jnp.float32),
        ],
        compiler_params=pltpu.CompilerParams(use_tc_tiling_on_sc=False),
    )
    return fn(h, enode2d, ecl2d, z2d, z1d, ones2d)


NCTX4 = N_ECLASSES * HIDDEN // 128
NACC4 = ACC_PAD * HIDDEN // 128


def _combine_body(acc_ref, cnt_ref, mt_ref, ctx_ref):
    a = acc_ref[0] + acc_ref[1]
    n4 = cnt_ref[0] + cnt_ref[1]
    inv4 = 1.0 / jnp.maximum(n4, 1.0)
    inv = jnp.dot(inv4, mt_ref[...], preferred_element_type=jnp.float32)
    ctx_ref[...] = a * inv


def _combine(acc4, cnt4):
    R = NACC4 // 4
    grid = (4,)
    MT = jnp.kron(jnp.eye(PACK, dtype=jnp.float32),
                  jnp.ones((1, HIDDEN), jnp.float32))
    return pl.pallas_call(
        _combine_body,
        grid=grid,
        in_specs=[
            pl.BlockSpec((NC, R, 128), lambda i: (0, i, 0)),
            pl.BlockSpec((NC, R, PACK), lambda i: (0, i, 0)),
            pl.BlockSpec((PACK, 128), lambda i: (0, 0)),
        ],
        out_specs=pl.BlockSpec((R, 128), lambda i: (i, 0)),
        out_shape=jax.ShapeDtypeStruct((NCTX4, 128), jnp.float32),
    )(acc4, cnt4, MT)


def kernel(embedding, eclass_ids, enode_ids, W1, b1, gamma, beta, W2, b2):
    emb4 = embedding.reshape(NP4, 128)
    h4, logit4 = _dense(emb4, W1, b1, gamma, beta, W2, b2)
    h = h4.reshape(N_ENODES, HIDDEN)
    enode2d = enode_ids.reshape(ROWS_TOTAL, CHUNK)
    ecl2d = eclass_ids.reshape(ROWS_TOTAL, CHUNK)
    z2d = jnp.zeros((ACC_STRIPE, HIDDEN), jnp.float32)
    z1d = jnp.zeros((ACC_STRIPE,), jnp.float32)
    ones2d = jnp.ones((CHUNK,), jnp.float32)
    acc, cnt = _segment_mean_partials(h, enode2d, ecl2d, z2d, z1d, ones2d)
    ctx4 = _combine(acc.reshape(NC, NACC4, 128),
                    cnt.reshape(NC, NACC4, PACK))
    return logit4.reshape(1, N_ENODES), ctx4.reshape(1, N_ECLASSES, HIDDEN)
